# initial kernel scaffold (unmeasured)
import functools

import jax
import jax.numpy as jnp
from jax import lax
from jax.experimental import pallas as pl
from jax.experimental.pallas import tpu as pltpu

N_DEV = 8
B = 2
SQ = 512
D_MODEL = 768
HEADS = 8
DH = 64
KV0 = 512
KV1 = 128
KV = KV0 + KV1
WIN = 128
CHUNK = SQ // N_DEV
MESH = pl.DeviceIdType.MESH


def kernel(x, Wq, K_ext, V_ext, Wo):
    def body(x_ref, wq_ref, k_ext_ref, v_ext_ref, wo_ref, out_ref,
             kbuf, vbuf, qbuf, ctxbuf, rsbuf,
             kv_recv_sems, kv_send_sems, local_sems,
             rs_send_sems, rs_recv_sems, ag_send_sems, ag_recv_sems):
        my = lax.axis_index("i")

        barrier_sem = pltpu.get_barrier_semaphore()
        for d in range(N_DEV):
            pl.semaphore_signal(barrier_sem, inc=1, device_id=(d,),
                                device_id_type=MESH)
        pl.semaphore_wait(barrier_sem, N_DEV)

        def kv_sends(src_dev):
            rows = KV0 if src_dev == 0 else KV1
            dst_lo = 0 if src_dev == 0 else KV0
            sem0 = 0 if src_dev == 0 else 2
            descs = []
            dests = [d for d in range(N_DEV) if d != src_dev]
            for i, d in enumerate(dests):
                for t, (src, dst) in enumerate(
                        [(k_ext_ref, kbuf), (v_ext_ref, vbuf)]):
                    descs.append(pltpu.make_async_remote_copy(
                        src_ref=src.at[:, pl.ds(0, rows), pl.ds(8 * d, 8), :],
                        dst_ref=dst.at[:, pl.ds(dst_lo, rows), :, :],
                        send_sem=kv_send_sems.at[2 * i + t],
                        recv_sem=kv_recv_sems.at[sem0 + t],
                        device_id=(d,), device_id_type=MESH,
                    ))
            return descs

        def kv_local_copies(src_dev):
            rows = KV0 if src_dev == 0 else KV1
            dst_lo = 0 if src_dev == 0 else KV0
            return [pltpu.make_async_copy(
                src.at[:, pl.ds(0, rows), pl.ds(8 * src_dev, 8), :],
                dst.at[:, pl.ds(dst_lo, rows), :, :],
                local_sems.at[t],
            ) for t, (src, dst) in enumerate(
                [(k_ext_ref, kbuf), (v_ext_ref, vbuf)])]

        for src_dev in (0, 1):
            @pl.when(my == src_dev)
            def _(src_dev=src_dev):
                for r in kv_sends(src_dev):
                    r.start()
                for c in kv_local_copies(src_dev):
                    c.start()

        for b in range(B):
            qbuf[b] = jnp.dot(x_ref[b], wq_ref[:],
                              preferred_element_type=jnp.float32)

        for src_dev in (0, 1):
            @pl.when(my == src_dev)
            def _(src_dev=src_dev):
                for c in kv_local_copies(src_dev):
                    c.wait()

            @pl.when(my != src_dev)
            def _(src_dev=src_dev):
                rows = KV0 if src_dev == 0 else KV1
                dst_lo = 0 if src_dev == 0 else KV0
                sem0 = 0 if src_dev == 0 else 2
                for t, buf in enumerate([kbuf, vbuf]):
                    pltpu.make_async_remote_copy(
                        src_ref=buf.at[:, pl.ds(dst_lo, rows), :, :],
                        dst_ref=buf.at[:, pl.ds(dst_lo, rows), :, :],
                        send_sem=kv_send_sems.at[t],
                        recv_sem=kv_recv_sems.at[sem0 + t],
                        device_id=(src_dev,), device_id_type=MESH,
                    ).wait_recv()

        qi = lax.broadcasted_iota(jnp.int32, (SQ, KV), 0)
        kj = lax.broadcasted_iota(jnp.int32, (SQ, KV), 1)
        mask = jnp.abs(qi - kj) <= WIN

        for b in range(B):
            for h in range(HEADS):
                qh = qbuf[b, :, pl.ds(DH * h, DH)]
                kh = kbuf[b, :, h, :]
                s = lax.dot_general(
                    qh, kh, (((1,), (1,)), ((), ())),
                    preferred_element_type=jnp.float32) * 0.125
                s = jnp.where(mask, s, -1e9)
                m = jnp.max(s, axis=1, keepdims=True)
                e = jnp.exp(s - m)
                w = e / jnp.sum(e, axis=1, keepdims=True)
                vh = vbuf[b, :, h, :]
                ctxbuf[:, pl.ds(DH * h, DH)] = jnp.dot(
                    w, vh, preferred_element_type=jnp.float32)
            out_ref[b] = jnp.dot(ctxbuf[:], wo_ref[:],
                                 preferred_element_type=jnp.float32)

        for src_dev in (0, 1):
            @pl.when(my == src_dev)
            def _(src_dev=src_dev):
                for r in kv_sends(src_dev):
                    r.wait_send()

        right = (my + 1) % N_DEV
        for s in range(N_DEV - 1):
            c_send = (my - s) % N_DEV
            c_recv = (my - s - 1) % N_DEV
            rdma = pltpu.make_async_remote_copy(
                src_ref=out_ref.at[:, pl.ds(c_send * CHUNK, CHUNK), :],
                dst_ref=rsbuf.at[s],
                send_sem=rs_send_sems.at[s],
                recv_sem=rs_recv_sems.at[s],
                device_id=(right,), device_id_type=MESH,
            )
            rdma.start()
            rdma.wait()
            sl = pl.ds(c_recv * CHUNK, CHUNK)
            out_ref[:, sl, :] = out_ref[:, sl, :] + rsbuf[s]

        for s in range(N_DEV - 1):
            c_send = (my + 1 - s) % N_DEV
            sl = pl.ds(c_send * CHUNK, CHUNK)
            rdma = pltpu.make_async_remote_copy(
                src_ref=out_ref.at[:, sl, :],
                dst_ref=out_ref.at[:, sl, :],
                send_sem=ag_send_sems.at[s],
                recv_sem=ag_recv_sems.at[s],
                device_id=(right,), device_id_type=MESH,
            )
            rdma.start()
            rdma.wait()

    return pl.pallas_call(
        body,
        out_shape=jax.ShapeDtypeStruct((B, SQ, D_MODEL), jnp.float32),
        in_specs=[
            pl.BlockSpec(memory_space=pltpu.VMEM),
            pl.BlockSpec(memory_space=pltpu.VMEM),
            pl.BlockSpec(memory_space=pltpu.ANY),
            pl.BlockSpec(memory_space=pltpu.ANY),
            pl.BlockSpec(memory_space=pltpu.VMEM),
        ],
        out_specs=pl.BlockSpec(memory_space=pltpu.VMEM),
        scratch_shapes=[
            pltpu.VMEM((B, KV, HEADS, DH), jnp.float32),
            pltpu.VMEM((B, KV, HEADS, DH), jnp.float32),
            pltpu.VMEM((B, SQ, HEADS * DH), jnp.float32),
            pltpu.VMEM((SQ, HEADS * DH), jnp.float32),
            pltpu.VMEM((N_DEV - 1, B, CHUNK, D_MODEL), jnp.float32),
            pltpu.SemaphoreType.DMA((4,)),
            pltpu.SemaphoreType.DMA((2 * (N_DEV - 1),)),
            pltpu.SemaphoreType.DMA((2,)),
            pltpu.SemaphoreType.DMA((N_DEV - 1,)),
            pltpu.SemaphoreType.DMA((N_DEV - 1,)),
            pltpu.SemaphoreType.DMA((N_DEV - 1,)),
            pltpu.SemaphoreType.DMA((N_DEV - 1,)),
        ],
        compiler_params=pltpu.CompilerParams(collective_id=0),
    )(x, Wq, K_ext, V_ext, Wo)


# baseline (device time: 525558 ns/iter reference)
import functools

import jax
import jax.numpy as jnp
from jax import lax
from jax.experimental import pallas as pl
from jax.experimental.pallas import tpu as pltpu

N_DEV = 8
B = 2
SQ = 512
D_MODEL = 768
HEADS = 8
DH = 64
KV0 = 512
KV1 = 128
KV = KV0 + KV1
WIN = 128
CHUNK = SQ // N_DEV
MESH = pl.DeviceIdType.MESH


def kernel(x, Wq, K_ext, V_ext, Wo):
    def body(x_ref, wq_ref, k_ext_ref, v_ext_ref, wo_ref, out_ref,
             kbuf, vbuf, qbuf, ctxbuf, rsbuf,
             kv_recv_sems, kv_send_sems, local_sems,
             rs_send_sems, rs_recv_sems, ag_send_sems, ag_recv_sems):
        my = lax.axis_index("i")

        barrier_sem = pltpu.get_barrier_semaphore()
        for d in range(N_DEV):
            pl.semaphore_signal(barrier_sem, inc=1, device_id=(d,),
                                device_id_type=MESH)
        pl.semaphore_wait(barrier_sem, N_DEV)

        def kv_sends(src_dev):
            rows = KV0 if src_dev == 0 else KV1
            dst_lo = 0 if src_dev == 0 else KV0
            sem0 = 0 if src_dev == 0 else 2
            descs = []
            dests = [d for d in range(N_DEV) if d != src_dev]
            for i, d in enumerate(dests):
                for t, (src, dst) in enumerate(
                        [(k_ext_ref, kbuf), (v_ext_ref, vbuf)]):
                    descs.append(pltpu.make_async_remote_copy(
                        src_ref=src.at[:, pl.ds(0, rows), pl.ds(8 * d, 8), :],
                        dst_ref=dst.at[:, pl.ds(dst_lo, rows), :, :],
                        send_sem=kv_send_sems.at[2 * i + t],
                        recv_sem=kv_recv_sems.at[sem0 + t],
                        device_id=(d,), device_id_type=MESH,
                    ))
            return descs

        def kv_local_copies(src_dev):
            rows = KV0 if src_dev == 0 else KV1
            dst_lo = 0 if src_dev == 0 else KV0
            return [pltpu.make_async_copy(
                src.at[:, pl.ds(0, rows), pl.ds(8 * src_dev, 8), :],
                dst.at[:, pl.ds(dst_lo, rows), :, :],
                local_sems.at[t],
            ) for t, (src, dst) in enumerate(
                [(k_ext_ref, kbuf), (v_ext_ref, vbuf)])]

        for src_dev in (0, 1):
            @pl.when(my == src_dev)
            def _(src_dev=src_dev):
                for r in kv_sends(src_dev):
                    r.start()
                for c in kv_local_copies(src_dev):
                    c.start()

        for b in range(B):
            qbuf[b] = jnp.dot(x_ref[b], wq_ref[:],
                              preferred_element_type=jnp.float32)

        for src_dev in (0, 1):
            @pl.when(my == src_dev)
            def _(src_dev=src_dev):
                for c in kv_local_copies(src_dev):
                    c.wait()

            @pl.when(my != src_dev)
            def _(src_dev=src_dev):
                rows = KV0 if src_dev == 0 else KV1
                dst_lo = 0 if src_dev == 0 else KV0
                sem0 = 0 if src_dev == 0 else 2
                for t, buf in enumerate([kbuf, vbuf]):
                    pltpu.make_async_remote_copy(
                        src_ref=buf.at[:, pl.ds(dst_lo, rows), :, :],
                        dst_ref=buf.at[:, pl.ds(dst_lo, rows), :, :],
                        send_sem=kv_send_sems.at[t],
                        recv_sem=kv_recv_sems.at[sem0 + t],
                        device_id=(src_dev,), device_id_type=MESH,
                    ).wait_recv()

        qi = lax.broadcasted_iota(jnp.int32, (SQ, KV), 0)
        kj = lax.broadcasted_iota(jnp.int32, (SQ, KV), 1)
        mask = jnp.abs(qi - kj) <= WIN

        for b in range(B):
            for h in range(HEADS):
                qh = qbuf[b, :, pl.ds(DH * h, DH)]
                kh = kbuf[b, :, h, :]
                s = lax.dot_general(
                    qh, kh, (((1,), (1,)), ((), ())),
                    preferred_element_type=jnp.float32) * 0.125
                s = jnp.where(mask, s, -1e9)
                m = jnp.max(s, axis=1, keepdims=True)
                e = jnp.exp(s - m)
                w = e / jnp.sum(e, axis=1, keepdims=True)
                vh = vbuf[b, :, h, :]
                ctxbuf[:, pl.ds(DH * h, DH)] = jnp.dot(
                    w, vh, preferred_element_type=jnp.float32)
            out_ref[b] = jnp.dot(ctxbuf[:], wo_ref[:],
                                 preferred_element_type=jnp.float32)

        for src_dev in (0, 1):
            @pl.when(my == src_dev)
            def _(src_dev=src_dev):
                for r in kv_sends(src_dev):
                    r.wait_send()

        right = (my + 1) % N_DEV
        for s in range(N_DEV - 1):
            c_send = (my - s) % N_DEV
            c_recv = (my - s - 1) % N_DEV
            rdma = pltpu.make_async_remote_copy(
                src_ref=out_ref.at[:, pl.ds(c_send * CHUNK, CHUNK), :],
                dst_ref=rsbuf.at[s],
                send_sem=rs_send_sems.at[s],
                recv_sem=rs_recv_sems.at[s],
                device_id=(right,), device_id_type=MESH,
            )
            rdma.start()
            rdma.wait()
            sl = pl.ds(c_recv * CHUNK, CHUNK)
            out_ref[:, sl, :] = out_ref[:, sl, :] + rsbuf[s]

        for s in range(N_DEV - 1):
            c_send = (my + 1 - s) % N_DEV
            sl = pl.ds(c_send * CHUNK, CHUNK)
            rdma = pltpu.make_async_remote_copy(
                src_ref=out_ref.at[:, sl, :],
                dst_ref=out_ref.at[:, sl, :],
                send_sem=ag_send_sems.at[s],
                recv_sem=ag_recv_sems.at[s],
                device_id=(right,), device_id_type=MESH,
            )
            rdma.start()
            rdma.wait()

    return pl.pallas_call(
        body,
        out_shape=jax.ShapeDtypeStruct((B, SQ, D_MODEL), jnp.float32),
        in_specs=[
            pl.BlockSpec(memory_space=pltpu.VMEM),
            pl.BlockSpec(memory_space=pltpu.VMEM),
            pl.BlockSpec(memory_space=pl.ANY),
            pl.BlockSpec(memory_space=pl.ANY),
            pl.BlockSpec(memory_space=pltpu.VMEM),
        ],
        out_specs=pl.BlockSpec(memory_space=pltpu.VMEM),
        scratch_shapes=[
            pltpu.VMEM((B, KV, HEADS, DH), jnp.float32),
            pltpu.VMEM((B, KV, HEADS, DH), jnp.float32),
            pltpu.VMEM((B, SQ, HEADS * DH), jnp.float32),
            pltpu.VMEM((SQ, HEADS * DH), jnp.float32),
            pltpu.VMEM((N_DEV - 1, B, CHUNK, D_MODEL), jnp.float32),
            pltpu.SemaphoreType.DMA((4,)),
            pltpu.SemaphoreType.DMA((2 * (N_DEV - 1),)),
            pltpu.SemaphoreType.DMA((2,)),
            pltpu.SemaphoreType.DMA((N_DEV - 1,)),
            pltpu.SemaphoreType.DMA((N_DEV - 1,)),
            pltpu.SemaphoreType.DMA((N_DEV - 1,)),
            pltpu.SemaphoreType.DMA((N_DEV - 1,)),
        ],
        compiler_params=pltpu.CompilerParams(collective_id=0),
    )(x, Wq, K_ext, V_ext, Wo)


# device time: 216810 ns/iter; 2.4240x vs baseline; 2.4240x over previous
import jax
import jax.numpy as jnp
from jax import lax
from jax.experimental import pallas as pl
from jax.experimental.pallas import tpu as pltpu

N_DEV = 8
B = 2
SQ = 512
D_MODEL = 768
HEADS = 8
DH = 64
HD = HEADS * DH
KV0 = 512
KV1 = 128
KV = KV0 + KV1
WIN = 128
MESH = pl.DeviceIdType.MESH


def kernel(x, Wq, K_ext, V_ext, Wo):
    def body(x_ref, wq_ref, k_ext_ref, v_ext_ref, wo_ref, out_ref,
             kbuf, vbuf, qbuf, ctxbuf, kcast, vcast, stage, rsbuf,
             kv_recv_sems, kv_send_sems, stage_sems,
             rs_send_sems, rs_recv_sems, ag_send_sems, ag_recv_sems):
        my = lax.axis_index("i")

        barrier_sem = pltpu.get_barrier_semaphore()
        for d in range(N_DEV):
            pl.semaphore_signal(barrier_sem, inc=1, device_id=(d,),
                                device_id_type=MESH)
        pl.semaphore_wait(barrier_sem, N_DEV)

        def scatter_params(src_dev):
            rows = KV0 if src_dev == 0 else KV1
            dst_lo = 0 if src_dev == 0 else KV0
            sem0 = 0 if src_dev == 0 else 2
            order = [d for d in range(N_DEV) if d != src_dev] + [src_dev]
            return rows, dst_lo, sem0, order

        def kv_sends(src_dev):
            rows, dst_lo, sem0, order = scatter_params(src_dev)
            descs = []
            for i, d in enumerate(order[:-1]):
                for t, (src, dst) in enumerate([(kcast, kbuf), (vcast, vbuf)]):
                    descs.append(pltpu.make_async_remote_copy(
                        src_ref=src.at[d, :, pl.ds(0, rows), :],
                        dst_ref=dst.at[:, pl.ds(dst_lo, rows), :],
                        send_sem=kv_send_sems.at[2 * i + t],
                        recv_sem=kv_recv_sems.at[sem0 + t],
                        device_id=(d,), device_id_type=MESH,
                    ))
            return descs

        for src_dev in (0, 1):
            @pl.when(my == src_dev)
            def _(src_dev=src_dev):
                rows, dst_lo, sem0, order = scatter_params(src_dev)
                sends = kv_sends(src_dev)

                def stage_copy(i):
                    d, slot = order[i], i % 2
                    return [pltpu.make_async_copy(
                        src.at[:, pl.ds(0, rows), pl.ds(HD * d, HD)],
                        stage.at[t, slot, :, pl.ds(0, rows), :],
                        stage_sems.at[2 * slot + t],
                    ) for t, src in enumerate([k_ext_ref, v_ext_ref])]

                for c in stage_copy(0) + stage_copy(1):
                    c.start()
                for i, d in enumerate(order):
                    slot = i % 2
                    for c in stage_copy(i):
                        c.wait()
                    kcast[d, :, pl.ds(0, rows), :] = stage[
                        0, slot, :, pl.ds(0, rows), :].astype(jnp.bfloat16)
                    vcast[d, :, pl.ds(0, rows), :] = stage[
                        1, slot, :, pl.ds(0, rows), :].astype(jnp.bfloat16)
                    if d == src_dev:
                        kbuf[:, pl.ds(dst_lo, rows), :] = kcast[
                            d, :, pl.ds(0, rows), :]
                        vbuf[:, pl.ds(dst_lo, rows), :] = vcast[
                            d, :, pl.ds(0, rows), :]
                    else:
                        sends[2 * i].start()
                        sends[2 * i + 1].start()
                    if i + 2 < len(order):
                        for c in stage_copy(i + 2):
                            c.start()

        for b in range(B):
            qbuf[b] = jnp.dot(x_ref[b], wq_ref[:],
                              preferred_element_type=jnp.float32)

        for src_dev in (0, 1):
            @pl.when(my != src_dev)
            def _(src_dev=src_dev):
                rows, dst_lo, sem0, _ = scatter_params(src_dev)
                for t, buf in enumerate([kbuf, vbuf]):
                    pltpu.make_async_remote_copy(
                        src_ref=buf.at[:, pl.ds(dst_lo, rows), :],
                        dst_ref=buf.at[:, pl.ds(dst_lo, rows), :],
                        send_sem=kv_send_sems.at[t],
                        recv_sem=kv_recv_sems.at[sem0 + t],
                        device_id=(src_dev,), device_id_type=MESH,
                    ).wait_recv()

        qi = lax.broadcasted_iota(jnp.int32, (SQ, KV), 0)
        kj = lax.broadcasted_iota(jnp.int32, (SQ, KV), 1)
        mask = jnp.abs(qi - kj) <= WIN

        for b in range(B):
            for h in range(HEADS):
                qh = qbuf[b, :, pl.ds(DH * h, DH)]
                kh = kbuf[b, :, pl.ds(DH * h, DH)].astype(jnp.float32)
                s = lax.dot_general(
                    qh, kh, (((1,), (1,)), ((), ())),
                    preferred_element_type=jnp.float32) * 0.125
                s = jnp.where(mask, s, -1e9)
                m = jnp.max(s, axis=1, keepdims=True)
                e = jnp.exp(s - m)
                w = e / jnp.sum(e, axis=1, keepdims=True)
                vh = vbuf[b, :, pl.ds(DH * h, DH)].astype(jnp.float32)
                ctxbuf[:, pl.ds(DH * h, DH)] = jnp.dot(
                    w, vh, preferred_element_type=jnp.float32)
            out_ref[b] = jnp.dot(ctxbuf[:], wo_ref[:],
                                 preferred_element_type=jnp.float32)

        for src_dev in (0, 1):
            @pl.when(my == src_dev)
            def _(src_dev=src_dev):
                for r in kv_sends(src_dev):
                    r.wait_send()

        p0, p1, p2 = my % 2, (my // 2) % 2, (my // 4) % 2

        base = 0
        for idx, (dist, size, bit, boff) in enumerate(
                [(1, 256, p0, 0), (2, 128, p1, 256), (4, 64, p2, 384)]):
            partner = my ^ dist
            send_off = base + (1 - bit) * size
            keep_off = base + bit * size
            rdma = pltpu.make_async_remote_copy(
                src_ref=out_ref.at[:, pl.ds(send_off, size), :],
                dst_ref=rsbuf.at[:, pl.ds(boff, size), :],
                send_sem=rs_send_sems.at[idx],
                recv_sem=rs_recv_sems.at[idx],
                device_id=(partner,), device_id_type=MESH,
            )
            rdma.start()
            rdma.wait()
            sl = pl.ds(keep_off, size)
            bl = pl.ds(boff, size)
            out_ref[:, sl, :] = out_ref[:, sl, :] + rsbuf[:, bl, :]
            base = keep_off

        cur = base
        for idx, (dist, size, bit) in enumerate(
                [(4, 64, p2), (2, 128, p1), (1, 256, p0)]):
            partner = my ^ dist
            sl = pl.ds(cur, size)
            rdma = pltpu.make_async_remote_copy(
                src_ref=out_ref.at[:, sl, :],
                dst_ref=out_ref.at[:, sl, :],
                send_sem=ag_send_sems.at[idx],
                recv_sem=ag_recv_sems.at[idx],
                device_id=(partner,), device_id_type=MESH,
            )
            rdma.start()
            rdma.wait()
            cur = cur - bit * size

    return pl.pallas_call(
        body,
        out_shape=jax.ShapeDtypeStruct((B, SQ, D_MODEL), jnp.float32),
        in_specs=[
            pl.BlockSpec(memory_space=pltpu.VMEM),
            pl.BlockSpec(memory_space=pltpu.VMEM),
            pl.BlockSpec(memory_space=pl.ANY),
            pl.BlockSpec(memory_space=pl.ANY),
            pl.BlockSpec(memory_space=pltpu.VMEM),
        ],
        out_specs=pl.BlockSpec(memory_space=pltpu.VMEM),
        scratch_shapes=[
            pltpu.VMEM((B, KV, HD), jnp.bfloat16),
            pltpu.VMEM((B, KV, HD), jnp.bfloat16),
            pltpu.VMEM((B, SQ, HD), jnp.float32),
            pltpu.VMEM((SQ, HD), jnp.float32),
            pltpu.VMEM((N_DEV, B, KV0, HD), jnp.bfloat16),
            pltpu.VMEM((N_DEV, B, KV0, HD), jnp.bfloat16),
            pltpu.VMEM((2, 2, B, KV0, HD), jnp.float32),
            pltpu.VMEM((B, SQ, D_MODEL), jnp.float32),
            pltpu.SemaphoreType.DMA((4,)),
            pltpu.SemaphoreType.DMA((2 * (N_DEV - 1),)),
            pltpu.SemaphoreType.DMA((4,)),
            pltpu.SemaphoreType.DMA((3,)),
            pltpu.SemaphoreType.DMA((3,)),
            pltpu.SemaphoreType.DMA((3,)),
            pltpu.SemaphoreType.DMA((3,)),
        ],
        compiler_params=pltpu.CompilerParams(
            collective_id=0, vmem_limit_bytes=100 * 1024 * 1024),
    )(x, Wq,
      K_ext.reshape(B, KV0, N_DEV * HD),
      V_ext.reshape(B, KV0, N_DEV * HD),
      Wo)


# device time: 187784 ns/iter; 2.7987x vs baseline; 1.1546x over previous
import jax
import jax.numpy as jnp
from jax import lax
from jax.experimental import pallas as pl
from jax.experimental.pallas import tpu as pltpu

N_DEV = 8
B = 2
SQ = 512
D_MODEL = 768
HEADS = 8
DH = 64
HD = HEADS * DH
KV0 = 512
KV1 = 128
KV = KV0 + KV1
WIN = 128
MESH = pl.DeviceIdType.MESH


def kernel(x, Wq, K_ext, V_ext, Wo):
    def body(x_ref, wq_ref, k_ext_ref, v_ext_ref, wo_ref, out_ref,
             kbuf, vbuf, qbuf, ctxbuf, wobuf, kcast, vcast, stage,
             rsbuf, rssnd, agbuf,
             kv_recv_sems, kv_send_sems, stage_sems,
             rs_send_sems, rs_recv_sems, ag_send_sems, ag_recv_sems):
        my = lax.axis_index("i")

        barrier_sem = pltpu.get_barrier_semaphore()
        for d in range(N_DEV):
            pl.semaphore_signal(barrier_sem, inc=1, device_id=(d,),
                                device_id_type=MESH)
        pl.semaphore_wait(barrier_sem, N_DEV)

        def scatter_params(src_dev):
            rows = KV0 if src_dev == 0 else KV1
            dst_lo = 0 if src_dev == 0 else KV0
            sem0 = 0 if src_dev == 0 else 2
            order = [d for d in range(N_DEV) if d != src_dev] + [src_dev]
            return rows, dst_lo, sem0, order

        def kv_sends(src_dev):
            rows, dst_lo, sem0, order = scatter_params(src_dev)
            descs = []
            for i, d in enumerate(order[:-1]):
                for t, (src, dst) in enumerate([(kcast, kbuf), (vcast, vbuf)]):
                    descs.append(pltpu.make_async_remote_copy(
                        src_ref=src.at[d, :, pl.ds(0, rows), :],
                        dst_ref=dst.at[:, pl.ds(dst_lo, rows), :],
                        send_sem=kv_send_sems.at[2 * i + t],
                        recv_sem=kv_recv_sems.at[sem0 + t],
                        device_id=(d,), device_id_type=MESH,
                    ))
            return descs

        for src_dev in (0, 1):
            @pl.when(my == src_dev)
            def _(src_dev=src_dev):
                rows, dst_lo, sem0, order = scatter_params(src_dev)
                sends = kv_sends(src_dev)

                def stage_copy(i):
                    d, slot = order[i], i % 2
                    return [pltpu.make_async_copy(
                        src.at[:, pl.ds(0, rows), pl.ds(HD * d, HD)],
                        stage.at[t, slot, :, pl.ds(0, rows), :],
                        stage_sems.at[2 * slot + t],
                    ) for t, src in enumerate([k_ext_ref, v_ext_ref])]

                for c in stage_copy(0) + stage_copy(1):
                    c.start()
                for i, d in enumerate(order):
                    slot = i % 2
                    for c in stage_copy(i):
                        c.wait()
                    kcast[d, :, pl.ds(0, rows), :] = stage[
                        0, slot, :, pl.ds(0, rows), :].astype(jnp.bfloat16)
                    vcast[d, :, pl.ds(0, rows), :] = stage[
                        1, slot, :, pl.ds(0, rows), :].astype(jnp.bfloat16)
                    if d == src_dev:
                        kbuf[:, pl.ds(dst_lo, rows), :] = kcast[
                            d, :, pl.ds(0, rows), :]
                        vbuf[:, pl.ds(dst_lo, rows), :] = vcast[
                            d, :, pl.ds(0, rows), :]
                    else:
                        sends[2 * i].start()
                        sends[2 * i + 1].start()
                    if i + 2 < len(order):
                        for c in stage_copy(i + 2):
                            c.start()

        for b in range(B):
            qbuf[b] = jnp.dot(
                x_ref[b], wq_ref[:],
                preferred_element_type=jnp.float32).astype(jnp.bfloat16)
        wobuf[:] = wo_ref[:].astype(jnp.bfloat16)

        for src_dev in (0, 1):
            @pl.when(my != src_dev)
            def _(src_dev=src_dev):
                rows, dst_lo, sem0, _ = scatter_params(src_dev)
                for t, buf in enumerate([kbuf, vbuf]):
                    pltpu.make_async_remote_copy(
                        src_ref=buf.at[:, pl.ds(dst_lo, rows), :],
                        dst_ref=buf.at[:, pl.ds(dst_lo, rows), :],
                        send_sem=kv_send_sems.at[t],
                        recv_sem=kv_recv_sems.at[sem0 + t],
                        device_id=(src_dev,), device_id_type=MESH,
                    ).wait_recv()

        qi = lax.broadcasted_iota(jnp.int32, (SQ, KV), 0)
        kj = lax.broadcasted_iota(jnp.int32, (SQ, KV), 1)
        mask = jnp.abs(qi - kj) <= WIN

        for b in range(B):
            for h in range(HEADS):
                qh = qbuf[b, :, pl.ds(DH * h, DH)]
                kh = kbuf[b, :, pl.ds(DH * h, DH)]
                s = lax.dot_general(
                    qh, kh, (((1,), (1,)), ((), ())),
                    preferred_element_type=jnp.float32) * 0.125
                s = jnp.where(mask, s, -1e9)
                m = jnp.max(s, axis=1, keepdims=True)
                e = jnp.exp(s - m)
                w = (e / jnp.sum(e, axis=1, keepdims=True)).astype(
                    jnp.bfloat16)
                vh = vbuf[b, :, pl.ds(DH * h, DH)]
                ctxbuf[:, pl.ds(DH * h, DH)] = jnp.dot(
                    w, vh, preferred_element_type=jnp.float32).astype(
                        jnp.bfloat16)
            out_ref[b] = jnp.dot(ctxbuf[:], wobuf[:],
                                 preferred_element_type=jnp.float32)

        for src_dev in (0, 1):
            @pl.when(my == src_dev)
            def _(src_dev=src_dev):
                for r in kv_sends(src_dev):
                    r.wait_send()

        p0, p1, p2 = my % 2, (my // 2) % 2, (my // 4) % 2

        base = 0
        for idx, (dist, size, bit, boff) in enumerate(
                [(1, 256, p0, 0), (2, 128, p1, 256), (4, 64, p2, 384)]):
            partner = my ^ dist
            send_off = base + (1 - bit) * size
            keep_off = base + bit * size
            rssnd[:, pl.ds(0, size), :] = out_ref[
                :, pl.ds(send_off, size), :].astype(jnp.bfloat16)
            rdma = pltpu.make_async_remote_copy(
                src_ref=rssnd.at[:, pl.ds(0, size), :],
                dst_ref=rsbuf.at[:, pl.ds(boff, size), :],
                send_sem=rs_send_sems.at[idx],
                recv_sem=rs_recv_sems.at[idx],
                device_id=(partner,), device_id_type=MESH,
            )
            rdma.start()
            rdma.wait()
            sl = pl.ds(keep_off, size)
            bl = pl.ds(boff, size)
            out_ref[:, sl, :] = out_ref[:, sl, :] + rsbuf[
                :, bl, :].astype(jnp.float32)
            base = keep_off

        agbuf[:, pl.ds(base, 64), :] = out_ref[
            :, pl.ds(base, 64), :].astype(jnp.bfloat16)
        cur = base
        for idx, (dist, size, bit) in enumerate(
                [(4, 64, p2), (2, 128, p1), (1, 256, p0)]):
            partner = my ^ dist
            sl = pl.ds(cur, size)
            rdma = pltpu.make_async_remote_copy(
                src_ref=agbuf.at[:, sl, :],
                dst_ref=agbuf.at[:, sl, :],
                send_sem=ag_send_sems.at[idx],
                recv_sem=ag_recv_sems.at[idx],
                device_id=(partner,), device_id_type=MESH,
            )
            rdma.start()
            rdma.wait()
            cur = cur - bit * size
        out_ref[:] = agbuf[:].astype(jnp.float32)

    return pl.pallas_call(
        body,
        out_shape=jax.ShapeDtypeStruct((B, SQ, D_MODEL), jnp.float32),
        in_specs=[
            pl.BlockSpec(memory_space=pltpu.VMEM),
            pl.BlockSpec(memory_space=pltpu.VMEM),
            pl.BlockSpec(memory_space=pl.ANY),
            pl.BlockSpec(memory_space=pl.ANY),
            pl.BlockSpec(memory_space=pltpu.VMEM),
        ],
        out_specs=pl.BlockSpec(memory_space=pltpu.VMEM),
        scratch_shapes=[
            pltpu.VMEM((B, KV, HD), jnp.bfloat16),
            pltpu.VMEM((B, KV, HD), jnp.bfloat16),
            pltpu.VMEM((B, SQ, HD), jnp.bfloat16),
            pltpu.VMEM((SQ, HD), jnp.bfloat16),
            pltpu.VMEM((SQ, D_MODEL), jnp.bfloat16),
            pltpu.VMEM((N_DEV, B, KV0, HD), jnp.bfloat16),
            pltpu.VMEM((N_DEV, B, KV0, HD), jnp.bfloat16),
            pltpu.VMEM((2, 2, B, KV0, HD), jnp.float32),
            pltpu.VMEM((B, SQ, D_MODEL), jnp.bfloat16),
            pltpu.VMEM((B, SQ // 2, D_MODEL), jnp.bfloat16),
            pltpu.VMEM((B, SQ, D_MODEL), jnp.bfloat16),
            pltpu.SemaphoreType.DMA((4,)),
            pltpu.SemaphoreType.DMA((2 * (N_DEV - 1),)),
            pltpu.SemaphoreType.DMA((4,)),
            pltpu.SemaphoreType.DMA((3,)),
            pltpu.SemaphoreType.DMA((3,)),
            pltpu.SemaphoreType.DMA((3,)),
            pltpu.SemaphoreType.DMA((3,)),
        ],
        compiler_params=pltpu.CompilerParams(
            collective_id=0, vmem_limit_bytes=100 * 1024 * 1024),
    )(x, Wq,
      K_ext.reshape(B, KV0, N_DEV * HD),
      V_ext.reshape(B, KV0, N_DEV * HD),
      Wo)


# device time: 166772 ns/iter; 3.1514x vs baseline; 1.1260x over previous
import jax
import jax.numpy as jnp
from jax import lax
from jax.experimental import pallas as pl
from jax.experimental.pallas import tpu as pltpu

N_DEV = 8
B = 2
SQ = 512
D_MODEL = 768
HEADS = 8
DH = 64
HD = HEADS * DH
KV0 = 512
KV1 = 128
KV = KV0 + KV1
WIN = 128
MESH = pl.DeviceIdType.MESH


def kernel(x, Wq, K_ext, V_ext, Wo):
    def body(x_ref, wq_ref, k_ext_ref, v_ext_ref, wo_ref, out_ref,
             kbuf, vbuf, qbuf, ctxbuf, wobuf, kcast, vcast, stage,
             rsbuf, rssnd, agbuf, rbuf,
             kv_recv_sems, kv_send_sems, stage_sems,
             relay_recv_sems, relay_send_sems,
             rs_send_sems, rs_recv_sems, ag_send_sems, ag_recv_sems):
        my = lax.axis_index("i")

        barrier_sem = pltpu.get_barrier_semaphore()
        for d in range(N_DEV):
            pl.semaphore_signal(barrier_sem, inc=1, device_id=(d,),
                                device_id_type=MESH)
        pl.semaphore_wait(barrier_sem, N_DEV)

        def scatter_params(src_dev):
            rows = KV0 if src_dev == 0 else KV1
            dst_lo = 0 if src_dev == 0 else KV0
            sem0 = 0 if src_dev == 0 else 2
            if src_dev == 0:
                order = [5, 1, 6, 3, 4, 2, 7, 0]
            else:
                order = [d for d in range(N_DEV) if d != 1] + [1]
            return rows, dst_lo, sem0, order

        RELAY = {(5, 0): (4, 0), (5, 1): (4, 1), (6, 0): (7, 0), (6, 1): (4, 2)}
        RELAY_FWD = {4: [(0, 5, 0), (1, 5, 1), (2, 6, 1)], 7: [(0, 6, 0)]}

        def kv_sends(src_dev):
            rows, dst_lo, sem0, order = scatter_params(src_dev)
            descs = []
            for i, d in enumerate(order[:-1]):
                for t, src in enumerate([kcast, vcast]):
                    if src_dev == 0 and (d, t) in RELAY:
                        node, slot = RELAY[(d, t)]
                        tgt = node
                        dst = rbuf.at[slot]
                        rsem = relay_recv_sems.at[slot]
                    else:
                        tgt = d
                        dst = (kbuf, vbuf)[t].at[:, pl.ds(dst_lo, rows), :]
                        rsem = kv_recv_sems.at[sem0 + t]
                    descs.append(pltpu.make_async_remote_copy(
                        src_ref=src.at[d, :, pl.ds(0, rows), :],
                        dst_ref=dst,
                        send_sem=kv_send_sems.at[2 * i + t],
                        recv_sem=rsem,
                        device_id=(tgt,), device_id_type=MESH,
                    ))
            return descs

        def relay_fwds(node):
            return [(slot, pltpu.make_async_remote_copy(
                src_ref=rbuf.at[slot],
                dst_ref=(kbuf, vbuf)[t].at[:, pl.ds(0, KV0), :],
                send_sem=relay_send_sems.at[slot],
                recv_sem=kv_recv_sems.at[t],
                device_id=(dest,), device_id_type=MESH,
            )) for slot, dest, t in RELAY_FWD[node]]

        for src_dev in (0, 1):
            @pl.when(my == src_dev)
            def _(src_dev=src_dev):
                rows, dst_lo, sem0, order = scatter_params(src_dev)
                sends = kv_sends(src_dev)

                def stage_copy(i):
                    d, slot = order[i], i % 2
                    return [pltpu.make_async_copy(
                        src.at[:, pl.ds(0, rows), pl.ds(HD * d, HD)],
                        stage.at[t, slot, :, pl.ds(0, rows), :],
                        stage_sems.at[2 * slot + t],
                    ) for t, src in enumerate([k_ext_ref, v_ext_ref])]

                for c in stage_copy(0) + stage_copy(1):
                    c.start()
                for i, d in enumerate(order):
                    slot = i % 2
                    for c in stage_copy(i):
                        c.wait()
                    kcast[d, :, pl.ds(0, rows), :] = stage[
                        0, slot, :, pl.ds(0, rows), :].astype(jnp.bfloat16)
                    vcast[d, :, pl.ds(0, rows), :] = stage[
                        1, slot, :, pl.ds(0, rows), :].astype(jnp.bfloat16)
                    if d == src_dev:
                        kbuf[:, pl.ds(dst_lo, rows), :] = kcast[
                            d, :, pl.ds(0, rows), :]
                        vbuf[:, pl.ds(dst_lo, rows), :] = vcast[
                            d, :, pl.ds(0, rows), :]
                    else:
                        sends[2 * i].start()
                        sends[2 * i + 1].start()
                    if i + 2 < len(order):
                        for c in stage_copy(i + 2):
                            c.start()

        for node in (4, 7):
            @pl.when(my == node)
            def _(node=node):
                for slot, fwd in relay_fwds(node):
                    pltpu.make_async_remote_copy(
                        src_ref=rbuf.at[slot], dst_ref=rbuf.at[slot],
                        send_sem=relay_send_sems.at[slot],
                        recv_sem=relay_recv_sems.at[slot],
                        device_id=(0,), device_id_type=MESH,
                    ).wait_recv()
                    fwd.start()

        for b in range(B):
            qbuf[b] = jnp.dot(
                x_ref[b], wq_ref[:],
                preferred_element_type=jnp.float32).astype(jnp.bfloat16)
        wobuf[:] = wo_ref[:].astype(jnp.bfloat16)

        for src_dev in (0, 1):
            @pl.when(my != src_dev)
            def _(src_dev=src_dev):
                rows, dst_lo, sem0, _ = scatter_params(src_dev)
                for t, buf in enumerate([kbuf, vbuf]):
                    pltpu.make_async_remote_copy(
                        src_ref=buf.at[:, pl.ds(dst_lo, rows), :],
                        dst_ref=buf.at[:, pl.ds(dst_lo, rows), :],
                        send_sem=kv_send_sems.at[t],
                        recv_sem=kv_recv_sems.at[sem0 + t],
                        device_id=(src_dev,), device_id_type=MESH,
                    ).wait_recv()

        qi = lax.broadcasted_iota(jnp.int32, (SQ, KV), 0)
        kj = lax.broadcasted_iota(jnp.int32, (SQ, KV), 1)
        mask = jnp.abs(qi - kj) <= WIN

        for b in range(B):
            for h in range(HEADS):
                qh = qbuf[b, :, pl.ds(DH * h, DH)]
                kh = kbuf[b, :, pl.ds(DH * h, DH)]
                s = lax.dot_general(
                    qh, kh, (((1,), (1,)), ((), ())),
                    preferred_element_type=jnp.float32) * 0.125
                s = jnp.where(mask, s, -1e9)
                m = jnp.max(s, axis=1, keepdims=True)
                e = jnp.exp(s - m)
                w = (e / jnp.sum(e, axis=1, keepdims=True)).astype(
                    jnp.bfloat16)
                vh = vbuf[b, :, pl.ds(DH * h, DH)]
                ctxbuf[:, pl.ds(DH * h, DH)] = jnp.dot(
                    w, vh, preferred_element_type=jnp.float32).astype(
                        jnp.bfloat16)
            out_ref[b] = jnp.dot(ctxbuf[:], wobuf[:],
                                 preferred_element_type=jnp.float32)

        for src_dev in (0, 1):
            @pl.when(my == src_dev)
            def _(src_dev=src_dev):
                for r in kv_sends(src_dev):
                    r.wait_send()
        for node in (4, 7):
            @pl.when(my == node)
            def _(node=node):
                for _slot, fwd in relay_fwds(node):
                    fwd.wait_send()

        p0, p1, p2 = my % 2, (my // 2) % 2, (my // 4) % 2

        base = 0
        for idx, (dist, size, bit, boff) in enumerate(
                [(1, 256, p0, 0), (2, 128, p1, 256), (4, 64, p2, 384)]):
            partner = my ^ dist
            send_off = base + (1 - bit) * size
            keep_off = base + bit * size
            rssnd[:, pl.ds(0, size), :] = out_ref[
                :, pl.ds(send_off, size), :].astype(jnp.bfloat16)
            rdma = pltpu.make_async_remote_copy(
                src_ref=rssnd.at[:, pl.ds(0, size), :],
                dst_ref=rsbuf.at[:, pl.ds(boff, size), :],
                send_sem=rs_send_sems.at[idx],
                recv_sem=rs_recv_sems.at[idx],
                device_id=(partner,), device_id_type=MESH,
            )
            rdma.start()
            rdma.wait()
            sl = pl.ds(keep_off, size)
            bl = pl.ds(boff, size)
            out_ref[:, sl, :] = out_ref[:, sl, :] + rsbuf[
                :, bl, :].astype(jnp.float32)
            base = keep_off

        agbuf[:, pl.ds(base, 64), :] = out_ref[
            :, pl.ds(base, 64), :].astype(jnp.bfloat16)
        cur = base
        for idx, (dist, size, bit) in enumerate(
                [(4, 64, p2), (2, 128, p1), (1, 256, p0)]):
            partner = my ^ dist
            sl = pl.ds(cur, size)
            rdma = pltpu.make_async_remote_copy(
                src_ref=agbuf.at[:, sl, :],
                dst_ref=agbuf.at[:, sl, :],
                send_sem=ag_send_sems.at[idx],
                recv_sem=ag_recv_sems.at[idx],
                device_id=(partner,), device_id_type=MESH,
            )
            rdma.start()
            rdma.wait()
            cur = cur - bit * size
        out_ref[:] = agbuf[:].astype(jnp.float32)

    return pl.pallas_call(
        body,
        out_shape=jax.ShapeDtypeStruct((B, SQ, D_MODEL), jnp.float32),
        in_specs=[
            pl.BlockSpec(memory_space=pltpu.VMEM),
            pl.BlockSpec(memory_space=pltpu.VMEM),
            pl.BlockSpec(memory_space=pl.ANY),
            pl.BlockSpec(memory_space=pl.ANY),
            pl.BlockSpec(memory_space=pltpu.VMEM),
        ],
        out_specs=pl.BlockSpec(memory_space=pltpu.VMEM),
        scratch_shapes=[
            pltpu.VMEM((B, KV, HD), jnp.bfloat16),
            pltpu.VMEM((B, KV, HD), jnp.bfloat16),
            pltpu.VMEM((B, SQ, HD), jnp.bfloat16),
            pltpu.VMEM((SQ, HD), jnp.bfloat16),
            pltpu.VMEM((SQ, D_MODEL), jnp.bfloat16),
            pltpu.VMEM((N_DEV, B, KV0, HD), jnp.bfloat16),
            pltpu.VMEM((N_DEV, B, KV0, HD), jnp.bfloat16),
            pltpu.VMEM((2, 2, B, KV0, HD), jnp.float32),
            pltpu.VMEM((B, SQ, D_MODEL), jnp.bfloat16),
            pltpu.VMEM((B, SQ // 2, D_MODEL), jnp.bfloat16),
            pltpu.VMEM((B, SQ, D_MODEL), jnp.bfloat16),
            pltpu.VMEM((3, B, KV0, HD), jnp.bfloat16),
            pltpu.SemaphoreType.DMA((4,)),
            pltpu.SemaphoreType.DMA((2 * (N_DEV - 1),)),
            pltpu.SemaphoreType.DMA((4,)),
            pltpu.SemaphoreType.DMA((3,)),
            pltpu.SemaphoreType.DMA((3,)),
            pltpu.SemaphoreType.DMA((3,)),
            pltpu.SemaphoreType.DMA((3,)),
            pltpu.SemaphoreType.DMA((3,)),
            pltpu.SemaphoreType.DMA((3,)),
        ],
        compiler_params=pltpu.CompilerParams(
            collective_id=0, vmem_limit_bytes=100 * 1024 * 1024),
    )(x, Wq,
      K_ext.reshape(B, KV0, N_DEV * HD),
      V_ext.reshape(B, KV0, N_DEV * HD),
      Wo)


# device time: 162925 ns/iter; 3.2258x vs baseline; 1.0236x over previous
import jax
import jax.numpy as jnp
from jax import lax
from jax.experimental import pallas as pl
from jax.experimental.pallas import tpu as pltpu

N_DEV = 8
B = 2
SQ = 512
D_MODEL = 768
HEADS = 8
DH = 64
HD = HEADS * DH
KV0 = 512
KV1 = 128
KV = KV0 + KV1
WIN = 128
MESH = pl.DeviceIdType.MESH


def kernel(x, Wq, K_ext, V_ext, Wo):
    def body(x_ref, wq_ref, k_ext_ref, v_ext_ref, wo_ref, out_ref,
             kbuf, vbuf, qbuf, ctxbuf, wobuf, kcast, vcast, stage,
             rsbuf, rssnd, agbuf, rbuf,
             kv_recv_sems, kv_send_sems, stage_sems,
             relay_recv_sems, relay_send_sems,
             rs_send_sems, rs_recv_sems, ag_send_sems, ag_recv_sems):
        my = lax.axis_index("i")

        barrier_sem = pltpu.get_barrier_semaphore()
        for d in range(N_DEV):
            pl.semaphore_signal(barrier_sem, inc=1, device_id=(d,),
                                device_id_type=MESH)
        pl.semaphore_wait(barrier_sem, N_DEV)

        def scatter_params(src_dev):
            rows = KV0 if src_dev == 0 else KV1
            dst_lo = 0 if src_dev == 0 else KV0
            sem0 = 0 if src_dev == 0 else 2
            if src_dev == 0:
                order = [5, 1, 6, 3, 4, 2, 7, 0]
            else:
                order = [d for d in range(N_DEV) if d != 1] + [1]
            return rows, dst_lo, sem0, order

        RELAY = {(5, 0): (4, 0), (5, 1): (4, 1), (6, 0): (7, 0), (6, 1): (4, 2)}
        RELAY_FWD = {4: [(0, 5, 0), (1, 5, 1), (2, 6, 1)], 7: [(0, 6, 0)]}

        def kv_sends(src_dev):
            rows, dst_lo, sem0, order = scatter_params(src_dev)
            descs = []
            for i, d in enumerate(order[:-1]):
                for t, src in enumerate([kcast, vcast]):
                    if src_dev == 0 and (d, t) in RELAY:
                        node, slot = RELAY[(d, t)]
                        tgt = node
                        dst = rbuf.at[slot]
                        rsem = relay_recv_sems.at[slot]
                    else:
                        tgt = d
                        dst = (kbuf, vbuf)[t].at[:, pl.ds(dst_lo, rows), :]
                        rsem = kv_recv_sems.at[sem0 + t]
                    descs.append(pltpu.make_async_remote_copy(
                        src_ref=src.at[d, :, pl.ds(0, rows), :],
                        dst_ref=dst,
                        send_sem=kv_send_sems.at[2 * i + t],
                        recv_sem=rsem,
                        device_id=(tgt,), device_id_type=MESH,
                    ))
            return descs

        def relay_fwds(node):
            return [(slot, pltpu.make_async_remote_copy(
                src_ref=rbuf.at[slot],
                dst_ref=(kbuf, vbuf)[t].at[:, pl.ds(0, KV0), :],
                send_sem=relay_send_sems.at[slot],
                recv_sem=kv_recv_sems.at[t],
                device_id=(dest,), device_id_type=MESH,
            )) for slot, dest, t in RELAY_FWD[node]]

        for src_dev in (0, 1):
            @pl.when(my == src_dev)
            def _(src_dev=src_dev):
                rows, dst_lo, sem0, order = scatter_params(src_dev)
                sends = kv_sends(src_dev)

                def stage_copy(i):
                    d, slot = order[i], i % 2
                    return [pltpu.make_async_copy(
                        src.at[:, pl.ds(0, rows), pl.ds(HD * d, HD)],
                        stage.at[t, slot, :, pl.ds(0, rows), :],
                        stage_sems.at[2 * slot + t],
                    ) for t, src in enumerate([k_ext_ref, v_ext_ref])]

                for c in stage_copy(0) + stage_copy(1):
                    c.start()
                for i, d in enumerate(order):
                    slot = i % 2
                    for c in stage_copy(i):
                        c.wait()
                    kcast[d, :, pl.ds(0, rows), :] = stage[
                        0, slot, :, pl.ds(0, rows), :].astype(jnp.bfloat16)
                    vcast[d, :, pl.ds(0, rows), :] = stage[
                        1, slot, :, pl.ds(0, rows), :].astype(jnp.bfloat16)
                    if d == src_dev:
                        kbuf[:, pl.ds(dst_lo, rows), :] = kcast[
                            d, :, pl.ds(0, rows), :]
                        vbuf[:, pl.ds(dst_lo, rows), :] = vcast[
                            d, :, pl.ds(0, rows), :]
                    else:
                        sends[2 * i].start()
                        sends[2 * i + 1].start()
                    if i + 2 < len(order):
                        for c in stage_copy(i + 2):
                            c.start()

        for node in (4, 7):
            @pl.when(my == node)
            def _(node=node):
                for slot, fwd in relay_fwds(node):
                    pltpu.make_async_remote_copy(
                        src_ref=rbuf.at[slot], dst_ref=rbuf.at[slot],
                        send_sem=relay_send_sems.at[slot],
                        recv_sem=relay_recv_sems.at[slot],
                        device_id=(0,), device_id_type=MESH,
                    ).wait_recv()
                    fwd.start()

        for b in range(B):
            qbuf[b] = jnp.dot(
                x_ref[b], wq_ref[:],
                preferred_element_type=jnp.float32).astype(jnp.bfloat16)
        wobuf[:] = wo_ref[:].astype(jnp.bfloat16)

        for src_dev in (0, 1):
            @pl.when(my != src_dev)
            def _(src_dev=src_dev):
                rows, dst_lo, sem0, _ = scatter_params(src_dev)
                for t, buf in enumerate([kbuf, vbuf]):
                    pltpu.make_async_remote_copy(
                        src_ref=buf.at[:, pl.ds(dst_lo, rows), :],
                        dst_ref=buf.at[:, pl.ds(dst_lo, rows), :],
                        send_sem=kv_send_sems.at[t],
                        recv_sem=kv_recv_sems.at[sem0 + t],
                        device_id=(src_dev,), device_id_type=MESH,
                    ).wait_recv()

        qi = lax.broadcasted_iota(jnp.int32, (SQ, KV), 0)
        kj = lax.broadcasted_iota(jnp.int32, (SQ, KV), 1)
        mask = jnp.abs(qi - kj) <= WIN

        def attn(b):
            for h in range(HEADS):
                qh = qbuf[b, :, pl.ds(DH * h, DH)]
                kh = kbuf[b, :, pl.ds(DH * h, DH)]
                s = lax.dot_general(
                    qh, kh, (((1,), (1,)), ((), ())),
                    preferred_element_type=jnp.float32) * 0.125
                s = jnp.where(mask, s, -1e9)
                m = jnp.max(s, axis=1, keepdims=True)
                e = jnp.exp(s - m)
                w = (e / jnp.sum(e, axis=1, keepdims=True)).astype(
                    jnp.bfloat16)
                vh = vbuf[b, :, pl.ds(DH * h, DH)]
                ctxbuf[:, pl.ds(DH * h, DH)] = jnp.dot(
                    w, vh, preferred_element_type=jnp.float32).astype(
                        jnp.bfloat16)
            out_ref[b] = jnp.dot(ctxbuf[:], wobuf[:],
                                 preferred_element_type=jnp.float32)

        p0, p1, p2 = my % 2, (my // 2) % 2, (my // 4) % 2
        RS = [(1, 256, p0, 0), (2, 128, p1, 256), (4, 64, p2, 384)]
        AG = [(4, 64, p2), (2, 128, p1), (1, 256, p0)]
        rs_base = [0, 0]
        ag_cur = [None, None]
        rs_inflight = {}
        ag_inflight = {}

        def rs_start(s, b):
            dist, size, bit, boff = RS[s]
            send_off = rs_base[b] + (1 - bit) * size
            rssnd[b, pl.ds(0, size), :] = out_ref[
                b, pl.ds(send_off, size), :].astype(jnp.bfloat16)
            rdma = pltpu.make_async_remote_copy(
                src_ref=rssnd.at[b, pl.ds(0, size), :],
                dst_ref=rsbuf.at[b, pl.ds(boff, size), :],
                send_sem=rs_send_sems.at[s, b],
                recv_sem=rs_recv_sems.at[s, b],
                device_id=(my ^ dist,), device_id_type=MESH,
            )
            rdma.start()
            rs_inflight[b] = rdma

        def rs_finish(s, b):
            dist, size, bit, boff = RS[s]
            rs_inflight[b].wait()
            keep_off = rs_base[b] + bit * size
            sl = pl.ds(keep_off, size)
            out_ref[b, sl, :] = out_ref[b, sl, :] + rsbuf[
                b, pl.ds(boff, size), :].astype(jnp.float32)
            rs_base[b] = keep_off

        def ag_start(s, b):
            dist, size, bit = AG[s]
            rdma = pltpu.make_async_remote_copy(
                src_ref=agbuf.at[b, pl.ds(ag_cur[b], size), :],
                dst_ref=agbuf.at[b, pl.ds(ag_cur[b], size), :],
                send_sem=ag_send_sems.at[s, b],
                recv_sem=ag_recv_sems.at[s, b],
                device_id=(my ^ dist,), device_id_type=MESH,
            )
            rdma.start()
            ag_inflight[b] = (rdma, ag_cur[b] - bit * size)

        def ag_finish(b):
            rdma, nxt = ag_inflight[b]
            rdma.wait()
            ag_cur[b] = nxt

        attn(0)
        rs_start(0, 0)
        attn(1)
        rs_start(0, 1)

        for src_dev in (0, 1):
            @pl.when(my == src_dev)
            def _(src_dev=src_dev):
                for r in kv_sends(src_dev):
                    r.wait_send()
        for node in (4, 7):
            @pl.when(my == node)
            def _(node=node):
                for _slot, fwd in relay_fwds(node):
                    fwd.wait_send()

        for s in range(3):
            for b in range(B):
                rs_finish(s, b)
                if s < 2:
                    rs_start(s + 1, b)
                else:
                    ag_cur[b] = rs_base[b]
                    agbuf[b, pl.ds(ag_cur[b], 64), :] = out_ref[
                        b, pl.ds(ag_cur[b], 64), :].astype(jnp.bfloat16)
                    ag_start(0, b)
        for s in range(3):
            for b in range(B):
                ag_finish(b)
                if s < 2:
                    ag_start(s + 1, b)
        out_ref[:] = agbuf[:].astype(jnp.float32)

    return pl.pallas_call(
        body,
        out_shape=jax.ShapeDtypeStruct((B, SQ, D_MODEL), jnp.float32),
        in_specs=[
            pl.BlockSpec(memory_space=pltpu.VMEM),
            pl.BlockSpec(memory_space=pltpu.VMEM),
            pl.BlockSpec(memory_space=pl.ANY),
            pl.BlockSpec(memory_space=pl.ANY),
            pl.BlockSpec(memory_space=pltpu.VMEM),
        ],
        out_specs=pl.BlockSpec(memory_space=pltpu.VMEM),
        scratch_shapes=[
            pltpu.VMEM((B, KV, HD), jnp.bfloat16),
            pltpu.VMEM((B, KV, HD), jnp.bfloat16),
            pltpu.VMEM((B, SQ, HD), jnp.bfloat16),
            pltpu.VMEM((SQ, HD), jnp.bfloat16),
            pltpu.VMEM((SQ, D_MODEL), jnp.bfloat16),
            pltpu.VMEM((N_DEV, B, KV0, HD), jnp.bfloat16),
            pltpu.VMEM((N_DEV, B, KV0, HD), jnp.bfloat16),
            pltpu.VMEM((2, 2, B, KV0, HD), jnp.float32),
            pltpu.VMEM((B, SQ, D_MODEL), jnp.bfloat16),
            pltpu.VMEM((B, SQ // 2, D_MODEL), jnp.bfloat16),
            pltpu.VMEM((B, SQ, D_MODEL), jnp.bfloat16),
            pltpu.VMEM((3, B, KV0, HD), jnp.bfloat16),
            pltpu.SemaphoreType.DMA((4,)),
            pltpu.SemaphoreType.DMA((2 * (N_DEV - 1),)),
            pltpu.SemaphoreType.DMA((4,)),
            pltpu.SemaphoreType.DMA((3,)),
            pltpu.SemaphoreType.DMA((3,)),
            pltpu.SemaphoreType.DMA((3, B)),
            pltpu.SemaphoreType.DMA((3, B)),
            pltpu.SemaphoreType.DMA((3, B)),
            pltpu.SemaphoreType.DMA((3, B)),
        ],
        compiler_params=pltpu.CompilerParams(
            collective_id=0, vmem_limit_bytes=100 * 1024 * 1024),
    )(x, Wq,
      K_ext.reshape(B, KV0, N_DEV * HD),
      V_ext.reshape(B, KV0, N_DEV * HD),
      Wo)


# device time: 153069 ns/iter; 3.4335x vs baseline; 1.0644x over previous
import jax
import jax.numpy as jnp
from jax import lax
from jax.experimental import pallas as pl
from jax.experimental.pallas import tpu as pltpu

N_DEV = 8
B = 2
SQ = 512
D_MODEL = 768
HEADS = 8
DH = 64
HD = HEADS * DH
KV0 = 512
KV1 = 128
KV = KV0 + KV1
WIN = 128
MESH = pl.DeviceIdType.MESH


def kernel(x, Wq, K_ext, V_ext, Wo):
    def body(x_ref, wq_ref, k_ext_ref, v_ext_ref, wo_ref, out_ref,
             kbuf, vbuf, qbuf, ctxbuf, wobuf, kcast, vcast, stage,
             rsbuf, rssnd, agbuf, rbuf,
             kv_recv_sems, kv_send_sems, stage_sems,
             relay_recv_sems, relay_send_sems,
             rs_send_sems, rs_recv_sems, ag_send_sems, ag_recv_sems):
        my = lax.axis_index("i")

        barrier_sem = pltpu.get_barrier_semaphore()
        for d in range(N_DEV):
            pl.semaphore_signal(barrier_sem, inc=1, device_id=(d,),
                                device_id_type=MESH)
        pl.semaphore_wait(barrier_sem, N_DEV)

        def scatter_params(src_dev):
            rows = KV0 if src_dev == 0 else KV1
            dst_lo = 0 if src_dev == 0 else KV0
            sem0 = 0 if src_dev == 0 else 2
            if src_dev == 0:
                order = [6, 5, 1, 3, 4, 2, 7, 0]
            else:
                order = [d for d in range(N_DEV) if d != 1] + [1]
            return rows, dst_lo, sem0, order

        RELAY = {(5, 0): (4, 0), (5, 1): (4, 1), (6, 0): (7, 0), (6, 1): (4, 2)}
        RELAY_FWD = {4: [(0, 5, 0), (1, 5, 1), (2, 6, 1)], 7: [(0, 6, 0)]}

        def kv_sends(src_dev):
            rows, dst_lo, sem0, order = scatter_params(src_dev)
            descs = []
            for i, d in enumerate(order[:-1]):
                for t, src in enumerate([kcast, vcast]):
                    if src_dev == 0 and (d, t) in RELAY:
                        node, slot = RELAY[(d, t)]
                        tgt = node
                        dst = rbuf.at[slot]
                        rsem = relay_recv_sems.at[slot]
                    else:
                        tgt = d
                        dst = (kbuf, vbuf)[t].at[:, pl.ds(dst_lo, rows), :]
                        rsem = kv_recv_sems.at[sem0 + t]
                    descs.append(pltpu.make_async_remote_copy(
                        src_ref=src.at[d, :, pl.ds(0, rows), :],
                        dst_ref=dst,
                        send_sem=kv_send_sems.at[2 * i + t],
                        recv_sem=rsem,
                        device_id=(tgt,), device_id_type=MESH,
                    ))
            return descs

        def relay_fwds(node):
            return [(slot, pltpu.make_async_remote_copy(
                src_ref=rbuf.at[slot],
                dst_ref=(kbuf, vbuf)[t].at[:, pl.ds(0, KV0), :],
                send_sem=relay_send_sems.at[slot],
                recv_sem=kv_recv_sems.at[t],
                device_id=(dest,), device_id_type=MESH,
            )) for slot, dest, t in RELAY_FWD[node]]

        for src_dev in (0, 1):
            @pl.when(my == src_dev)
            def _(src_dev=src_dev):
                rows, dst_lo, sem0, order = scatter_params(src_dev)
                sends = kv_sends(src_dev)

                def stage_copy(i):
                    d, slot = order[i], i % 2
                    return [pltpu.make_async_copy(
                        src.at[:, pl.ds(0, rows), pl.ds(HD * d, HD)],
                        stage.at[t, slot, :, pl.ds(0, rows), :],
                        stage_sems.at[2 * slot + t],
                    ) for t, src in enumerate([k_ext_ref, v_ext_ref])]

                for c in stage_copy(0) + stage_copy(1):
                    c.start()
                for i, d in enumerate(order):
                    slot = i % 2
                    for c in stage_copy(i):
                        c.wait()
                    kcast[d, :, pl.ds(0, rows), :] = stage[
                        0, slot, :, pl.ds(0, rows), :].astype(jnp.bfloat16)
                    vcast[d, :, pl.ds(0, rows), :] = stage[
                        1, slot, :, pl.ds(0, rows), :].astype(jnp.bfloat16)
                    if d == src_dev:
                        kbuf[:, pl.ds(dst_lo, rows), :] = kcast[
                            d, :, pl.ds(0, rows), :]
                        vbuf[:, pl.ds(dst_lo, rows), :] = vcast[
                            d, :, pl.ds(0, rows), :]
                    else:
                        sends[2 * i].start()
                        sends[2 * i + 1].start()
                    if i + 2 < len(order):
                        for c in stage_copy(i + 2):
                            c.start()

        for node in (4, 7):
            @pl.when(my == node)
            def _(node=node):
                for slot, fwd in relay_fwds(node):
                    pltpu.make_async_remote_copy(
                        src_ref=rbuf.at[slot], dst_ref=rbuf.at[slot],
                        send_sem=relay_send_sems.at[slot],
                        recv_sem=relay_recv_sems.at[slot],
                        device_id=(0,), device_id_type=MESH,
                    ).wait_recv()
                    fwd.start()

        for b in range(B):
            qbuf[b] = jnp.dot(
                x_ref[b], wq_ref[:],
                preferred_element_type=jnp.float32).astype(jnp.bfloat16)
        wobuf[:] = wo_ref[:].astype(jnp.bfloat16)

        for src_dev in (0, 1):
            @pl.when(my != src_dev)
            def _(src_dev=src_dev):
                rows, dst_lo, sem0, _ = scatter_params(src_dev)
                for t, buf in enumerate([kbuf, vbuf]):
                    pltpu.make_async_remote_copy(
                        src_ref=buf.at[:, pl.ds(dst_lo, rows), :],
                        dst_ref=buf.at[:, pl.ds(dst_lo, rows), :],
                        send_sem=kv_send_sems.at[t],
                        recv_sem=kv_recv_sems.at[sem0 + t],
                        device_id=(src_dev,), device_id_type=MESH,
                    ).wait_recv()

        qi = lax.broadcasted_iota(jnp.int32, (SQ, KV), 0)
        kj = lax.broadcasted_iota(jnp.int32, (SQ, KV), 1)
        mask = jnp.abs(qi - kj) <= WIN

        def attn(b):
            for h in range(HEADS):
                qh = qbuf[b, :, pl.ds(DH * h, DH)]
                kh = kbuf[b, :, pl.ds(DH * h, DH)]
                s = lax.dot_general(
                    qh, kh, (((1,), (1,)), ((), ())),
                    preferred_element_type=jnp.float32) * 0.125
                s = jnp.where(mask, s, -1e9)
                m = jnp.max(s, axis=1, keepdims=True)
                e = jnp.exp(s - m)
                w = (e / jnp.sum(e, axis=1, keepdims=True)).astype(
                    jnp.bfloat16)
                vh = vbuf[b, :, pl.ds(DH * h, DH)]
                ctxbuf[:, pl.ds(DH * h, DH)] = jnp.dot(
                    w, vh, preferred_element_type=jnp.float32).astype(
                        jnp.bfloat16)
            out_ref[b] = jnp.dot(ctxbuf[:], wobuf[:],
                                 preferred_element_type=jnp.float32)

        p0, p1, p2 = my % 2, (my // 2) % 2, (my // 4) % 2
        RS = [(1, 256, p0, 0), (4, 128, p2, 256), (2, 64, p1, 384)]
        AG = [(2, 64, p1), (4, 128, p2), (1, 256, p0)]
        rs_base = [0, 0]
        ag_cur = [None, None]
        rs_inflight = {}
        ag_inflight = {}

        def rs_start(s, b):
            dist, size, bit, boff = RS[s]
            send_off = rs_base[b] + (1 - bit) * size
            rssnd[b, pl.ds(0, size), :] = out_ref[
                b, pl.ds(send_off, size), :].astype(jnp.bfloat16)
            rdma = pltpu.make_async_remote_copy(
                src_ref=rssnd.at[b, pl.ds(0, size), :],
                dst_ref=rsbuf.at[b, pl.ds(boff, size), :],
                send_sem=rs_send_sems.at[s, b],
                recv_sem=rs_recv_sems.at[s, b],
                device_id=(my ^ dist,), device_id_type=MESH,
            )
            rdma.start()
            rs_inflight[b] = rdma

        def rs_finish(s, b):
            dist, size, bit, boff = RS[s]
            rs_inflight[b].wait()
            keep_off = rs_base[b] + bit * size
            sl = pl.ds(keep_off, size)
            out_ref[b, sl, :] = out_ref[b, sl, :] + rsbuf[
                b, pl.ds(boff, size), :].astype(jnp.float32)
            rs_base[b] = keep_off

        def ag_start(s, b):
            dist, size, bit = AG[s]
            rdma = pltpu.make_async_remote_copy(
                src_ref=agbuf.at[b, pl.ds(ag_cur[b], size), :],
                dst_ref=agbuf.at[b, pl.ds(ag_cur[b], size), :],
                send_sem=ag_send_sems.at[s, b],
                recv_sem=ag_recv_sems.at[s, b],
                device_id=(my ^ dist,), device_id_type=MESH,
            )
            rdma.start()
            ag_inflight[b] = (rdma, ag_cur[b] - bit * size)

        def ag_finish(b):
            rdma, nxt = ag_inflight[b]
            rdma.wait()
            ag_cur[b] = nxt

        attn(0)
        rs_start(0, 0)
        attn(1)
        rs_start(0, 1)

        for src_dev in (0, 1):
            @pl.when(my == src_dev)
            def _(src_dev=src_dev):
                for r in kv_sends(src_dev):
                    r.wait_send()
        for node in (4, 7):
            @pl.when(my == node)
            def _(node=node):
                for _slot, fwd in relay_fwds(node):
                    fwd.wait_send()

        for s in range(3):
            for b in range(B):
                rs_finish(s, b)
                if s < 2:
                    rs_start(s + 1, b)
                else:
                    ag_cur[b] = rs_base[b]
                    agbuf[b, pl.ds(ag_cur[b], 64), :] = out_ref[
                        b, pl.ds(ag_cur[b], 64), :].astype(jnp.bfloat16)
                    ag_start(0, b)
        for s in range(3):
            for b in range(B):
                ag_finish(b)
                if s < 2:
                    ag_start(s + 1, b)
        out_ref[:] = agbuf[:].astype(jnp.float32)

    return pl.pallas_call(
        body,
        out_shape=jax.ShapeDtypeStruct((B, SQ, D_MODEL), jnp.float32),
        in_specs=[
            pl.BlockSpec(memory_space=pltpu.VMEM),
            pl.BlockSpec(memory_space=pltpu.VMEM),
            pl.BlockSpec(memory_space=pl.ANY),
            pl.BlockSpec(memory_space=pl.ANY),
            pl.BlockSpec(memory_space=pltpu.VMEM),
        ],
        out_specs=pl.BlockSpec(memory_space=pltpu.VMEM),
        scratch_shapes=[
            pltpu.VMEM((B, KV, HD), jnp.bfloat16),
            pltpu.VMEM((B, KV, HD), jnp.bfloat16),
            pltpu.VMEM((B, SQ, HD), jnp.bfloat16),
            pltpu.VMEM((SQ, HD), jnp.bfloat16),
            pltpu.VMEM((SQ, D_MODEL), jnp.bfloat16),
            pltpu.VMEM((N_DEV, B, KV0, HD), jnp.bfloat16),
            pltpu.VMEM((N_DEV, B, KV0, HD), jnp.bfloat16),
            pltpu.VMEM((2, 2, B, KV0, HD), jnp.float32),
            pltpu.VMEM((B, SQ, D_MODEL), jnp.bfloat16),
            pltpu.VMEM((B, SQ // 2, D_MODEL), jnp.bfloat16),
            pltpu.VMEM((B, SQ, D_MODEL), jnp.bfloat16),
            pltpu.VMEM((3, B, KV0, HD), jnp.bfloat16),
            pltpu.SemaphoreType.DMA((4,)),
            pltpu.SemaphoreType.DMA((2 * (N_DEV - 1),)),
            pltpu.SemaphoreType.DMA((4,)),
            pltpu.SemaphoreType.DMA((3,)),
            pltpu.SemaphoreType.DMA((3,)),
            pltpu.SemaphoreType.DMA((3, B)),
            pltpu.SemaphoreType.DMA((3, B)),
            pltpu.SemaphoreType.DMA((3, B)),
            pltpu.SemaphoreType.DMA((3, B)),
        ],
        compiler_params=pltpu.CompilerParams(
            collective_id=0, vmem_limit_bytes=100 * 1024 * 1024),
    )(x, Wq,
      K_ext.reshape(B, KV0, N_DEV * HD),
      V_ext.reshape(B, KV0, N_DEV * HD),
      Wo)
